# baseline (speedup 1.0000x reference)
import functools

import jax
import jax.numpy as jnp
from jax.experimental import pallas as pl
from jax.experimental.pallas import tpu as pltpu

G, IN_DIM, OUT_DIM, COLS = 4, 4096, 4096, 256
BK = 1024
NK = IN_DIM // BK
NS = 4
QO = OUT_DIM // NS


def _body(x_ref, w0_ref, w1_ref, w2_ref, w3_ref, b_ref, o_ref):
    k = pl.program_id(1)
    x = x_ref[0]
    dn = (((0,), (0,)), ((), ()))
    w_refs = (w0_ref, w1_ref, w2_ref, w3_ref)
    accs = [jax.lax.dot_general(w[0], x, dimension_numbers=dn,
                                preferred_element_type=jnp.float32)
            for w in w_refs]

    @pl.when(k == 0)
    def _():
        bias = b_ref[0, 0].reshape(OUT_DIM, 1)
        for i, acc in enumerate(accs):
            o_ref[0, i * QO:(i + 1) * QO] = acc + bias[i * QO:(i + 1) * QO]

    @pl.when(k > 0)
    def _():
        for i, acc in enumerate(accs):
            o_ref[0, i * QO:(i + 1) * QO] += acc


@functools.partial(jax.jit, static_argnames=("interpret",))
def kernel(inputs, W, b, interpret=False):
    w_spec = lambda i: pl.BlockSpec((1, BK, QO), lambda g, k, i=i: (g, k, i))
    return pl.pallas_call(
        _body,
        grid=(G, NK),
        in_specs=[
            pl.BlockSpec((1, BK, COLS), lambda g, k: (g, k, 0)),
            w_spec(0), w_spec(1), w_spec(2), w_spec(3),
            pl.BlockSpec((1, 1, OUT_DIM), lambda g, k: (g, 0, 0)),
        ],
        out_specs=pl.BlockSpec((1, OUT_DIM, COLS), lambda g, k: (g, 0, 0)),
        out_shape=jax.ShapeDtypeStruct((G, OUT_DIM, COLS), jnp.float32),
        compiler_params=pltpu.CompilerParams(
            dimension_semantics=("parallel", "arbitrary"),
            vmem_limit_bytes=100 * 1024 * 1024,
        ),
        interpret=interpret,
    )(inputs, W, W, W, W, b.reshape(G, 1, OUT_DIM))
